# Initial kernel scaffold; baseline (speedup 1.0000x reference)
import jax
import jax.numpy as jnp
from jax.experimental import pallas as pl


def kernel(image_embedding, image_pe, point_embedding, params):
    raise NotImplementedError("write your pallas kernel here")


# speedup vs baseline: 1.3884x; 1.3884x over previous
import functools
import math

import jax
import jax.numpy as jnp
from jax.experimental import pallas as pl
from jax.experimental.pallas import tpu as pltpu

_HEADS = 8


def _lin(x, p):
    w = p['w'][...]
    y = jax.lax.dot_general(x, w, (((1,), (1,)), ((), ())),
                            preferred_element_type=jnp.float32)
    return y + p['b'][...]


def _ln(x, p):
    m = jnp.mean(x, axis=-1, keepdims=True)
    xc = x - m
    v = jnp.mean(xc * xc, axis=-1, keepdims=True)
    return xc * jax.lax.rsqrt(v + 1e-5) * p['g'][...] + p['b'][...]


def _attention(p, q_in, k_in, v_in):
    q = _lin(q_in, p['q'])
    k = _lin(k_in, p['k'])
    v = _lin(v_in, p['v'])
    nq, C = q.shape
    nk = k.shape[0]
    hd = C // _HEADS
    scale = 1.0 / math.sqrt(hd)
    lane = jax.lax.broadcasted_iota(jnp.int32, (1, C), 1)
    out = jnp.zeros((nq, C), jnp.float32)
    for h in range(_HEADS):
        mask = ((lane >= h * hd) & (lane < (h + 1) * hd)).astype(jnp.float32)
        if nq <= nk:
            lhs, rhs = q * mask, k
        else:
            lhs, rhs = q, k * mask
        logits = jax.lax.dot_general(lhs, rhs, (((1,), (1,)), ((), ())),
                                     preferred_element_type=jnp.float32)
        logits = logits * scale
        mx = jnp.max(logits, axis=-1, keepdims=True)
        e = jnp.exp(logits - mx)
        a = e * (1.0 / jnp.sum(e, axis=-1, keepdims=True))
        if nq <= nk:
            o = jax.lax.dot_general(a, v, (((1,), (0,)), ((), ())),
                                    preferred_element_type=jnp.float32)
            out = out + o * mask
        else:
            o = jax.lax.dot_general(a, v * mask, (((1,), (0,)), ((), ())),
                                    preferred_element_type=jnp.float32)
            out = out + o
    return _lin(out, p['o'])


def _body(treedef, n_param, *refs):
    keys_ref, kpe_ref, point_ref = refs[:3]
    param_refs = refs[3:3 + n_param]
    q_out_ref, k_out_ref = refs[3 + n_param:]
    p = jax.tree_util.tree_unflatten(treedef, list(param_refs))

    keys = keys_ref[0]
    kpe = kpe_ref[0]
    point = point_ref[0]
    queries = point
    for i, bp in enumerate(p['blocks']):
        if i == 0:
            queries = _attention(bp['self_attn'], queries, queries, queries)
        else:
            qq = queries + point
            queries = queries + _attention(bp['self_attn'], qq, qq, queries)
        queries = _ln(queries, bp['norm1'])
        qq = queries + point
        kk = keys + kpe
        queries = queries + _attention(bp['cross_t2i'], qq, kk, keys)
        queries = _ln(queries, bp['norm2'])
        h1 = jnp.maximum(_lin(queries, bp['mlp']['lin1']), 0.0)
        queries = queries + _lin(h1, bp['mlp']['lin2'])
        queries = _ln(queries, bp['norm3'])
        qq = queries + point
        kk = keys + kpe
        keys = keys + _attention(bp['cross_i2t'], kk, qq, queries)
        keys = _ln(keys, bp['norm4'])
    qq = queries + point
    kk = keys + kpe
    queries = queries + _attention(p['final_attn'], qq, kk, keys)
    queries = _ln(queries, p['norm_final'])
    q_out_ref[0] = queries
    k_out_ref[0] = keys


@jax.jit
def kernel(image_embedding, image_pe, point_embedding, params):
    bs, c, h, w = image_embedding.shape
    n = h * w
    npt = point_embedding.shape[1]
    keys0 = image_embedding.reshape(bs, c, n).transpose(0, 2, 1)
    kpe0 = image_pe.reshape(bs, c, n).transpose(0, 2, 1)

    flat, treedef = jax.tree_util.tree_flatten(params)
    flat = [f.reshape(1, -1) if f.ndim == 1 else f for f in flat]

    data_specs = [
        pl.BlockSpec((1, n, c), lambda b: (b, 0, 0)),
        pl.BlockSpec((1, n, c), lambda b: (b, 0, 0)),
        pl.BlockSpec((1, npt, c), lambda b: (b, 0, 0)),
    ]
    w_specs = [
        pl.BlockSpec(f.shape, lambda b, nd=f.ndim: (0,) * nd) for f in flat
    ]
    out_specs = [
        pl.BlockSpec((1, npt, c), lambda b: (b, 0, 0)),
        pl.BlockSpec((1, n, c), lambda b: (b, 0, 0)),
    ]
    out_shape = [
        jax.ShapeDtypeStruct((bs, npt, c), jnp.float32),
        jax.ShapeDtypeStruct((bs, n, c), jnp.float32),
    ]
    body = functools.partial(_body, treedef, len(flat))
    qs, ks = pl.pallas_call(
        body,
        grid=(bs,),
        in_specs=data_specs + w_specs,
        out_specs=out_specs,
        out_shape=out_shape,
        compiler_params=pltpu.CompilerParams(
            dimension_semantics=("arbitrary",),
        ),
    )(keys0, kpe0, point_embedding, *flat)
    return qs, ks
